# Initial kernel scaffold; baseline (speedup 1.0000x reference)
import jax
import jax.numpy as jnp
from jax.experimental import pallas as pl


def kernel(x, edge_index, W, b, u):
    raise NotImplementedError("write your pallas kernel here")


# speedup vs baseline: 37.2235x; 37.2235x over previous
import dataclasses
import functools

import jax
import jax.numpy as jnp
from jax import lax
from jax.experimental import pallas as pl
from jax.experimental.pallas import tpu as pltpu
from jax.experimental.pallas import tpu_sc as plsc

L = 16
NSUB = 16
NSC = 2
NW = NSC * NSUB
CHUNK = 128


def _tc_matmul_body(n, n_pad, x_ref, w_ref, wt_ref, u_ref, h_ref):
    eps = 1e-12
    w = w_ref[...]
    v = jnp.dot(u_ref[...], w, precision=lax.Precision.HIGHEST,
                preferred_element_type=jnp.float32)
    v = v / (jnp.sqrt(jnp.sum(v * v)) + eps)
    wv = jnp.dot(v, wt_ref[...], precision=lax.Precision.HIGHEST,
                 preferred_element_type=jnp.float32)
    un = wv / (jnp.sqrt(jnp.sum(wv * wv)) + eps)
    sigma = jnp.sum(un * wv)
    h = jnp.dot(x_ref[...], w, precision=lax.Precision.HIGHEST,
                preferred_element_type=jnp.float32)
    h_ref[pl.ds(0, n), :] = h * (1.0 / sigma)
    h_ref[pl.ds(n, n_pad - n), :] = jnp.zeros((n_pad - n, h.shape[1]),
                                              jnp.float32)


def _newton_rsqrt(d):
    y = plsc.bitcast(jnp.int32(0x5F3759DF) - (plsc.bitcast(d, jnp.int32) >> 1),
                     jnp.float32)
    for _ in range(3):
        y = y * (1.5 - 0.5 * d * y * y)
    return y


def _sc_body(n_pad, cp, erows, hrows_pt, arows_pt,
             h_hbm, src_hbm, dst_hbm, acc_hbm, dinv_hbm, g_hbm,
             out_sp, deg_sp,
             dsth_buf, srcbuf, dstbuf, hbuf, rows_buf, deg_buf, dinv_buf,
             zrow_buf, ones_buf):
    cid = lax.axis_index("c")
    sid = lax.axis_index("s")
    wid = cid * NSUB + sid
    rpt = n_pad // NSUB
    r0 = sid * rpt

    zv = jnp.zeros((L,), jnp.float32)

    @pl.loop(0, CHUNK)
    def _(r):
        for j in range(cp // L):
            rows_buf[r, pl.ds(j * L, L)] = zv

    @pl.loop(0, rpt // L)
    def _(i):
        zrow_buf[pl.ds(i * L, L)] = zv

    @pl.loop(0, CHUNK // L)
    def _(i):
        ones_buf[pl.ds(i * L, L)] = zv + 1.0

    for k in range(rpt // CHUNK):
        pltpu.sync_copy(rows_buf, out_sp.at[pl.ds(r0 + k * CHUNK, CHUNK), :])
    pltpu.sync_copy(zrow_buf, deg_sp.at[pl.ds(r0, rpt)])
    plsc.subcore_barrier()

    pltpu.sync_copy(dst_hbm.at[pl.ds(sid * hrows_pt, hrows_pt), :], dsth_buf)

    @pl.loop(0, hrows_pt)
    def _(c):
        pltpu.sync_copy(ones_buf, deg_sp.at[dsth_buf.at[c]], add=True)

    plsc.subcore_barrier()

    pltpu.sync_copy(deg_sp.at[pl.ds(r0, rpt)], deg_buf)

    @pl.loop(0, rpt // L)
    def _(i):
        d = deg_buf[pl.ds(i * L, L)] + 1.0
        dinv_buf[pl.ds(i * L, L)] = _newton_rsqrt(d)

    @pl.when(cid == 0)
    def _():
        pltpu.sync_copy(dinv_buf, dinv_hbm.at[pl.ds(r0, rpt)])

    pltpu.sync_copy(h_hbm.at[pl.ds(r0, rpt), :], hbuf)

    @pl.loop(0, rpt // L)
    def _(i):
        dvec = dinv_buf[pl.ds(i * L, L)]
        for k in range(L):
            s = dvec[k]
            r = i * L + k
            for j in range(cp // L):
                hbuf[r, pl.ds(j * L, L)] = hbuf[r, pl.ds(j * L, L)] * s

    pltpu.sync_copy(hbuf, g_hbm.at[cid].at[pl.ds(r0, rpt), :])

    pltpu.sync_copy(src_hbm.at[pl.ds(wid * arows_pt, arows_pt), :], srcbuf)
    pltpu.sync_copy(dst_hbm.at[pl.ds(wid * arows_pt, arows_pt), :], dstbuf)
    plsc.subcore_barrier()

    @pl.loop(0, arows_pt)
    def _(c):
        pltpu.sync_copy(g_hbm.at[cid].at[srcbuf.at[c]], rows_buf)
        pltpu.sync_copy(rows_buf, out_sp.at[dstbuf.at[c]], add=True)

    plsc.subcore_barrier()

    for k in range(rpt // CHUNK):
        sl = pl.ds(r0 + k * CHUNK, CHUNK)
        pltpu.sync_copy(out_sp.at[sl, :], rows_buf)
        pltpu.sync_copy(rows_buf, acc_hbm.at[cid, sl, :])


def _tc_combine_body(acc_ref, dinv_ref, h_ref, b_ref, o_ref):
    dv = dinv_ref[...]
    o_ref[...] = (dv * (acc_ref[0] + acc_ref[1])
                  + (dv * dv) * h_ref[...] + b_ref[...])


def _sc_compiler_params():
    cp = pltpu.CompilerParams()
    fields = pltpu.CompilerParams.__dataclass_fields__
    if "needs_layout_passes" in fields:
        cp = dataclasses.replace(cp, needs_layout_passes=False)
    if "use_tc_tiling_on_sc" in fields:
        cp = dataclasses.replace(cp, use_tc_tiling_on_sc=False)
    return cp


def kernel(x, edge_index, W, b, u):
    n, d_in = x.shape
    c_out = W.shape[1]
    e = edge_index.shape[1]

    cp = ((c_out + L - 1) // L) * L
    n_pad = ((n + CHUNK + 1023) // 1024) * 1024
    erows = ((e // CHUNK + 255) // 256) * 256
    e_pad = erows * CHUNK
    hrows_pt = erows // NSUB
    arows_pt = erows // NW

    w_p = jnp.pad(W, ((0, 0), (0, cp - c_out)))
    b_p = jnp.pad(b, (0, cp - c_out)).reshape(1, cp)
    u_r = u.reshape(1, d_in)

    pad_idx = n + (jnp.arange(e_pad - e, dtype=jnp.int32) % CHUNK)
    src2d = jnp.concatenate([edge_index[0], pad_idx]).reshape(erows, CHUNK)
    dst2d = jnp.concatenate([edge_index[1], pad_idx]).reshape(erows, CHUNK)

    h = pl.pallas_call(
        functools.partial(_tc_matmul_body, n, n_pad),
        out_shape=jax.ShapeDtypeStruct((n_pad, cp), jnp.float32),
    )(x, w_p, w_p.T, u_r)

    sc_fn = pl.kernel(
        functools.partial(_sc_body, n_pad, cp, erows, hrows_pt, arows_pt),
        out_type=(jax.ShapeDtypeStruct((NSC, n_pad, cp), jnp.float32),
                  jax.ShapeDtypeStruct((n_pad,), jnp.float32),
                  jax.ShapeDtypeStruct((NSC, n_pad, cp), jnp.float32)),
        mesh=plsc.VectorSubcoreMesh(core_axis_name="c", subcore_axis_name="s"),
        compiler_params=_sc_compiler_params(),
        scratch_types=[
            pltpu.VMEM_SHARED((n_pad, cp), jnp.float32),
            pltpu.VMEM_SHARED((n_pad,), jnp.float32),
            pltpu.VMEM((hrows_pt, CHUNK), jnp.int32),
            pltpu.VMEM((arows_pt, CHUNK), jnp.int32),
            pltpu.VMEM((arows_pt, CHUNK), jnp.int32),
            pltpu.VMEM((n_pad // NSUB, cp), jnp.float32),
            pltpu.VMEM((CHUNK, cp), jnp.float32),
            pltpu.VMEM((n_pad // NSUB,), jnp.float32),
            pltpu.VMEM((n_pad // NSUB,), jnp.float32),
            pltpu.VMEM((n_pad // NSUB,), jnp.float32),
            pltpu.VMEM((CHUNK,), jnp.float32),
        ],
    )
    acc, dinv, _g = sc_fn(h, src2d, dst2d)

    out_full = pl.pallas_call(
        _tc_combine_body,
        out_shape=jax.ShapeDtypeStruct((n_pad, cp), jnp.float32),
    )(acc, dinv.reshape(n_pad, 1), h, b_p)

    return out_full[:n, :c_out]


# speedup vs baseline: 48.5978x; 1.3056x over previous
import dataclasses
import functools

import jax
import jax.numpy as jnp
from jax import lax
from jax.experimental import pallas as pl
from jax.experimental.pallas import tpu as pltpu
from jax.experimental.pallas import tpu_sc as plsc

L = 16
NSUB = 16
NSC = 2
NW = NSC * NSUB
CHUNK = 128


def _tc_matmul_body(n, n_pad, x_ref, w_ref, wt_ref, u_ref, h_ref):
    eps = 1e-12
    w = w_ref[...]
    v = jnp.dot(u_ref[...], w, precision=lax.Precision.HIGHEST,
                preferred_element_type=jnp.float32)
    v = v / (jnp.sqrt(jnp.sum(v * v)) + eps)
    wv = jnp.dot(v, wt_ref[...], precision=lax.Precision.HIGHEST,
                 preferred_element_type=jnp.float32)
    un = wv / (jnp.sqrt(jnp.sum(wv * wv)) + eps)
    sigma = jnp.sum(un * wv)
    h = jnp.dot(x_ref[...], w, precision=lax.Precision.HIGHEST,
                preferred_element_type=jnp.float32)
    h_ref[pl.ds(0, n), :] = h * (1.0 / sigma)
    h_ref[pl.ds(n, n_pad - n), :] = jnp.zeros((n_pad - n, h.shape[1]),
                                              jnp.float32)


def _newton_rsqrt(d):
    y = plsc.bitcast(jnp.int32(0x5F3759DF) - (plsc.bitcast(d, jnp.int32) >> 1),
                     jnp.float32)
    for _ in range(3):
        y = y * (1.5 - 0.5 * d * y * y)
    return y


def _sc_body(n_pad, cp, erows, hrows_pt, arows_pt,
             h_hbm, src_hbm, dst_hbm, acc_hbm, dinv_hbm, g_hbm,
             out_sp, deg_sp,
             dsth_buf, srcbuf, dstbuf, hbuf, rows_buf, rows_buf1,
             deg_buf, dinv_buf, zrow_buf, ones_buf,
             sem_dsth, sem_src, sem_dst, sem_h, sem_hist,
             sem_g0, sem_g1, sem_s0, sem_s1):
    cid = lax.axis_index("c")
    sid = lax.axis_index("s")
    wid = cid * NSUB + sid
    rpt = n_pad // NSUB
    r0 = sid * rpt

    h_desc = pltpu.async_copy(h_hbm.at[pl.ds(r0, rpt), :], hbuf, sem_h)
    dsth_desc = pltpu.async_copy(dst_hbm.at[pl.ds(sid * hrows_pt, hrows_pt), :],
                                 dsth_buf, sem_dsth)

    zv = jnp.zeros((L,), jnp.float32)

    @pl.loop(0, CHUNK)
    def _(r):
        for j in range(cp // L):
            rows_buf[r, pl.ds(j * L, L)] = zv

    @pl.loop(0, rpt // L)
    def _(i):
        zrow_buf[pl.ds(i * L, L)] = zv

    @pl.loop(0, CHUNK // L)
    def _(i):
        ones_buf[pl.ds(i * L, L)] = zv + 1.0

    for k in range(rpt // CHUNK):
        pltpu.sync_copy(rows_buf, out_sp.at[pl.ds(r0 + k * CHUNK, CHUNK), :])
    pltpu.sync_copy(zrow_buf, deg_sp.at[pl.ds(r0, rpt)])
    dsth_desc.wait()
    plsc.subcore_barrier()

    hist_k = 8

    @pl.loop(0, hist_k)
    def _(c):
        pltpu.async_copy(ones_buf, deg_sp.at[dsth_buf.at[c]], sem_hist,
                         add=True)

    @pl.loop(hist_k, hrows_pt)
    def _(c):
        pltpu.make_async_copy(ones_buf, deg_sp.at[dsth_buf.at[0]],
                              sem_hist).wait()
        pltpu.async_copy(ones_buf, deg_sp.at[dsth_buf.at[c]], sem_hist,
                         add=True)

    @pl.loop(0, hist_k)
    def _(c):
        pltpu.make_async_copy(ones_buf, deg_sp.at[dsth_buf.at[0]],
                              sem_hist).wait()

    plsc.subcore_barrier()

    pltpu.sync_copy(deg_sp.at[pl.ds(r0, rpt)], deg_buf)

    @pl.loop(0, rpt // L)
    def _(i):
        d = deg_buf[pl.ds(i * L, L)] + 1.0
        dinv_buf[pl.ds(i * L, L)] = _newton_rsqrt(d)

    @pl.when(cid == 0)
    def _():
        pltpu.sync_copy(dinv_buf, dinv_hbm.at[pl.ds(r0, rpt)])

    h_desc.wait()

    @pl.loop(0, rpt // L)
    def _(i):
        dvec = dinv_buf[pl.ds(i * L, L)]
        for k in range(L):
            s = dvec[k]
            r = i * L + k
            for j in range(cp // L):
                hbuf[r, pl.ds(j * L, L)] = hbuf[r, pl.ds(j * L, L)] * s

    pltpu.sync_copy(hbuf, g_hbm.at[cid].at[pl.ds(r0, rpt), :])

    pltpu.sync_copy(src_hbm.at[pl.ds(wid * arows_pt, arows_pt), :], srcbuf)
    pltpu.sync_copy(dst_hbm.at[pl.ds(wid * arows_pt, arows_pt), :], dstbuf)
    plsc.subcore_barrier()

    def g_start(c, buf, sem):
        pltpu.async_copy(g_hbm.at[cid].at[srcbuf.at[c]], buf, sem)

    def g_wait(buf, sem):
        pltpu.make_async_copy(g_hbm.at[cid].at[srcbuf.at[0]], buf, sem).wait()

    def s_start(c, buf, sem):
        pltpu.async_copy(buf, out_sp.at[dstbuf.at[c]], sem, add=True)

    def s_wait(buf, sem):
        pltpu.make_async_copy(buf, out_sp.at[dstbuf.at[0]], sem).wait()

    def g_start(c, buf, sem):
        pltpu.async_copy(g_hbm.at[cid].at[srcbuf.at[c]], buf, sem)

    def g_wait(buf, sem):
        pltpu.make_async_copy(g_hbm.at[cid].at[srcbuf.at[0]], buf, sem).wait()

    def s_start(c, buf, sem):
        pltpu.async_copy(buf, out_sp.at[dstbuf.at[c]], sem, add=True)

    def s_wait(buf, sem):
        pltpu.make_async_copy(buf, out_sp.at[dstbuf.at[0]], sem).wait()

    g_start(0, rows_buf, sem_g0)
    g_start(1, rows_buf1, sem_g1)

    @pl.loop(0, arows_pt // 2 - 1)
    def _(i):
        c0 = 2 * i
        g_wait(rows_buf, sem_g0)
        s_start(c0, rows_buf, sem_s0)
        g_wait(rows_buf1, sem_g1)
        s_start(c0 + 1, rows_buf1, sem_s1)
        s_wait(rows_buf, sem_s0)
        g_start(c0 + 2, rows_buf, sem_g0)
        s_wait(rows_buf1, sem_s1)
        g_start(c0 + 3, rows_buf1, sem_g1)

    g_wait(rows_buf, sem_g0)
    s_start(arows_pt - 2, rows_buf, sem_s0)
    g_wait(rows_buf1, sem_g1)
    s_start(arows_pt - 1, rows_buf1, sem_s1)
    s_wait(rows_buf, sem_s0)
    s_wait(rows_buf1, sem_s1)

    plsc.subcore_barrier()

    for k in range(rpt // CHUNK):
        sl = pl.ds(r0 + k * CHUNK, CHUNK)
        pltpu.sync_copy(out_sp.at[sl, :], rows_buf)
        pltpu.sync_copy(rows_buf, acc_hbm.at[cid, sl, :])


def _tc_combine_body(acc_ref, dinv_ref, h_ref, b_ref, o_ref):
    dv = dinv_ref[...]
    o_ref[...] = (dv * (acc_ref[0] + acc_ref[1])
                  + (dv * dv) * h_ref[...] + b_ref[...])


def _sc_compiler_params():
    cp = pltpu.CompilerParams()
    fields = pltpu.CompilerParams.__dataclass_fields__
    if "needs_layout_passes" in fields:
        cp = dataclasses.replace(cp, needs_layout_passes=False)
    if "use_tc_tiling_on_sc" in fields:
        cp = dataclasses.replace(cp, use_tc_tiling_on_sc=False)
    return cp


def kernel(x, edge_index, W, b, u):
    n, d_in = x.shape
    c_out = W.shape[1]
    e = edge_index.shape[1]

    cp = ((c_out + L - 1) // L) * L
    n_pad = ((n + CHUNK + 1023) // 1024) * 1024
    erows = ((e // CHUNK + 255) // 256) * 256
    e_pad = erows * CHUNK
    hrows_pt = erows // NSUB
    arows_pt = erows // NW

    w_p = jnp.pad(W, ((0, 0), (0, cp - c_out)))
    b_p = jnp.pad(b, (0, cp - c_out)).reshape(1, cp)
    u_r = u.reshape(1, d_in)

    pad_idx = n + (jnp.arange(e_pad - e, dtype=jnp.int32) % CHUNK)
    src2d = jnp.concatenate([edge_index[0], pad_idx]).reshape(erows, CHUNK)
    dst2d = jnp.concatenate([edge_index[1], pad_idx]).reshape(erows, CHUNK)

    h = pl.pallas_call(
        functools.partial(_tc_matmul_body, n, n_pad),
        out_shape=jax.ShapeDtypeStruct((n_pad, cp), jnp.float32),
    )(x, w_p, w_p.T, u_r)

    sc_fn = pl.kernel(
        functools.partial(_sc_body, n_pad, cp, erows, hrows_pt, arows_pt),
        out_type=(jax.ShapeDtypeStruct((NSC, n_pad, cp), jnp.float32),
                  jax.ShapeDtypeStruct((n_pad,), jnp.float32),
                  jax.ShapeDtypeStruct((NSC, n_pad, cp), jnp.float32)),
        mesh=plsc.VectorSubcoreMesh(core_axis_name="c", subcore_axis_name="s"),
        compiler_params=_sc_compiler_params(),
        scratch_types=[
            pltpu.VMEM_SHARED((n_pad, cp), jnp.float32),
            pltpu.VMEM_SHARED((n_pad,), jnp.float32),
            pltpu.VMEM((hrows_pt, CHUNK), jnp.int32),
            pltpu.VMEM((arows_pt, CHUNK), jnp.int32),
            pltpu.VMEM((arows_pt, CHUNK), jnp.int32),
            pltpu.VMEM((n_pad // NSUB, cp), jnp.float32),
            pltpu.VMEM((CHUNK, cp), jnp.float32),
            pltpu.VMEM((CHUNK, cp), jnp.float32),
            pltpu.VMEM((n_pad // NSUB,), jnp.float32),
            pltpu.VMEM((n_pad // NSUB,), jnp.float32),
            pltpu.VMEM((n_pad // NSUB,), jnp.float32),
            pltpu.VMEM((CHUNK,), jnp.float32),
            pltpu.SemaphoreType.DMA,
            pltpu.SemaphoreType.DMA,
            pltpu.SemaphoreType.DMA,
            pltpu.SemaphoreType.DMA,
            pltpu.SemaphoreType.DMA,
            pltpu.SemaphoreType.DMA,
            pltpu.SemaphoreType.DMA,
            pltpu.SemaphoreType.DMA,
            pltpu.SemaphoreType.DMA,
        ],
    )
    acc, dinv, _g = sc_fn(h, src2d, dst2d)

    out_full = pl.pallas_call(
        _tc_combine_body,
        out_shape=jax.ShapeDtypeStruct((n_pad, cp), jnp.float32),
    )(acc, dinv.reshape(n_pad, 1), h, b_p)

    return out_full[:n, :c_out]


# speedup vs baseline: 48.6066x; 1.0002x over previous
import dataclasses
import functools

import jax
import jax.numpy as jnp
from jax import lax
from jax.experimental import pallas as pl
from jax.experimental.pallas import tpu as pltpu
from jax.experimental.pallas import tpu_sc as plsc

L = 16
NSUB = 16
NSC = 2
NW = NSC * NSUB
CHUNK = 128


def _tc_matmul_body(n, n_pad, x_ref, w_ref, wt_ref, u_ref, h_ref):
    eps = 1e-12
    w = w_ref[...]
    v = jnp.dot(u_ref[...], w, precision=lax.Precision.HIGHEST,
                preferred_element_type=jnp.float32)
    v = v / (jnp.sqrt(jnp.sum(v * v)) + eps)
    wv = jnp.dot(v, wt_ref[...], precision=lax.Precision.HIGHEST,
                 preferred_element_type=jnp.float32)
    un = wv / (jnp.sqrt(jnp.sum(wv * wv)) + eps)
    sigma = jnp.sum(un * wv)
    h = jnp.dot(x_ref[...], w, precision=lax.Precision.HIGHEST,
                preferred_element_type=jnp.float32)
    h_ref[pl.ds(0, n), :] = h * (1.0 / sigma)
    h_ref[pl.ds(n, n_pad - n), :] = jnp.zeros((n_pad - n, h.shape[1]),
                                              jnp.float32)


def _newton_rsqrt(d):
    y = plsc.bitcast(jnp.int32(0x5F3759DF) - (plsc.bitcast(d, jnp.int32) >> 1),
                     jnp.float32)
    for _ in range(3):
        y = y * (1.5 - 0.5 * d * y * y)
    return y


def _sc_body(n_pad, cp, erows,
             h_hbm, edges_hbm, b_hbm, acc_hbm, g_hbm,
             out_sp, deg_sp,
             dsth_buf, srcbuf, dstbuf, hbuf, rows_buf, rows_buf1,
             deg_buf, dinv_buf, zrow_buf, ones_buf, b_buf,
             sem_dsth, sem_src, sem_dst, sem_h, sem_hist,
             sem_g0, sem_g1, sem_s0, sem_s1):
    cid = lax.axis_index("c")
    sid = lax.axis_index("s")
    wid = cid * NSUB + sid
    rpt = n_pad // NSUB
    r0 = sid * rpt

    hbase, hrem = erows // NSUB, erows % NSUB
    abase, arem = erows // NW, erows % NW
    hrow0 = (sid * hbase + jnp.minimum(sid, hrem)) * CHUNK
    arow0 = (wid * abase + jnp.minimum(wid, arem)) * CHUNK
    n_hist = hbase + jnp.where(sid < hrem, 1, 0)

    h_desc = pltpu.async_copy(h_hbm.at[pl.ds(r0, rpt), :], hbuf, sem_h)
    dsth_desc = pltpu.async_copy(
        edges_hbm.at[1, pl.ds(hrow0, hbase * CHUNK)],
        dsth_buf.at[pl.ds(0, hbase * CHUNK)], sem_dsth)

    @pl.when(sid < hrem)
    def _():
        pltpu.sync_copy(edges_hbm.at[1, pl.ds(hrow0 + hbase * CHUNK, CHUNK)],
                        dsth_buf.at[pl.ds(hbase * CHUNK, CHUNK)])

    zv = jnp.zeros((L,), jnp.float32)

    @pl.loop(0, CHUNK)
    def _(r):
        for j in range(cp // L):
            rows_buf[r, pl.ds(j * L, L)] = zv

    @pl.loop(0, rpt // L)
    def _(i):
        zrow_buf[pl.ds(i * L, L)] = zv

    @pl.loop(0, CHUNK // L)
    def _(i):
        ones_buf[pl.ds(i * L, L)] = zv + 1.0

    for k in range(rpt // CHUNK):
        pltpu.sync_copy(rows_buf, out_sp.at[pl.ds(r0 + k * CHUNK, CHUNK), :])
    pltpu.sync_copy(zrow_buf, deg_sp.at[pl.ds(r0, rpt)])
    pltpu.sync_copy(b_hbm, b_buf)
    dsth_desc.wait()
    plsc.subcore_barrier()

    hist_k = 8

    @pl.loop(0, hist_k)
    def _(c):
        pltpu.async_copy(ones_buf, deg_sp.at[dsth_buf.at[pl.ds(c * CHUNK,
                                                               CHUNK)]],
                         sem_hist, add=True)

    @pl.loop(hist_k, n_hist)
    def _(c):
        pltpu.make_async_copy(ones_buf, deg_sp.at[dsth_buf.at[pl.ds(0, CHUNK)]],
                              sem_hist).wait()
        pltpu.async_copy(ones_buf, deg_sp.at[dsth_buf.at[pl.ds(c * CHUNK,
                                                               CHUNK)]],
                         sem_hist, add=True)

    @pl.loop(0, hist_k)
    def _(c):
        pltpu.make_async_copy(ones_buf, deg_sp.at[dsth_buf.at[pl.ds(0, CHUNK)]],
                              sem_hist).wait()

    plsc.subcore_barrier()

    pltpu.sync_copy(deg_sp.at[pl.ds(r0, rpt)], deg_buf)

    @pl.loop(0, rpt // L)
    def _(i):
        d = deg_buf[pl.ds(i * L, L)] + 1.0
        dinv_buf[pl.ds(i * L, L)] = _newton_rsqrt(d)

    h_desc.wait()

    @pl.loop(0, rpt // L)
    def _(i):
        dvec = dinv_buf[pl.ds(i * L, L)]
        for k in range(L):
            s = dvec[k]
            r = i * L + k
            for j in range(cp // L):
                hbuf[r, pl.ds(j * L, L)] = hbuf[r, pl.ds(j * L, L)] * s

    pltpu.sync_copy(hbuf, g_hbm.at[cid].at[pl.ds(r0, rpt), :])

    src_desc = pltpu.async_copy(edges_hbm.at[0, pl.ds(arow0, abase * CHUNK)],
                                srcbuf.at[pl.ds(0, abase * CHUNK)], sem_src)
    dst_desc = pltpu.async_copy(edges_hbm.at[1, pl.ds(arow0, abase * CHUNK)],
                                dstbuf.at[pl.ds(0, abase * CHUNK)], sem_dst)

    @pl.when(wid < arem)
    def _():
        pltpu.sync_copy(edges_hbm.at[0, pl.ds(arow0 + abase * CHUNK, CHUNK)],
                        srcbuf.at[pl.ds(abase * CHUNK, CHUNK)])
        pltpu.sync_copy(edges_hbm.at[1, pl.ds(arow0 + abase * CHUNK, CHUNK)],
                        dstbuf.at[pl.ds(abase * CHUNK, CHUNK)])

    src_desc.wait()
    dst_desc.wait()
    plsc.subcore_barrier()

    def src_idx(c):
        return srcbuf.at[pl.ds(c * CHUNK, CHUNK)]

    def dst_idx(c):
        return dstbuf.at[pl.ds(c * CHUNK, CHUNK)]

    def g_start(c, buf, sem):
        pltpu.async_copy(g_hbm.at[cid].at[src_idx(c)], buf, sem)

    def g_wait(buf, sem):
        pltpu.make_async_copy(g_hbm.at[cid].at[src_idx(0)], buf, sem).wait()

    def s_start(c, buf, sem):
        pltpu.async_copy(buf, out_sp.at[dst_idx(c)], sem, add=True)

    def s_wait(buf, sem):
        pltpu.make_async_copy(buf, out_sp.at[dst_idx(0)], sem).wait()

    g_start(0, rows_buf, sem_g0)
    g_start(1, rows_buf1, sem_g1)

    @pl.loop(0, abase // 2 - 1)
    def _(i):
        c0 = 2 * i
        g_wait(rows_buf, sem_g0)
        s_start(c0, rows_buf, sem_s0)
        g_wait(rows_buf1, sem_g1)
        s_start(c0 + 1, rows_buf1, sem_s1)
        s_wait(rows_buf, sem_s0)
        g_start(c0 + 2, rows_buf, sem_g0)
        s_wait(rows_buf1, sem_s1)
        g_start(c0 + 3, rows_buf1, sem_g1)

    g_wait(rows_buf, sem_g0)
    s_start(abase - 2, rows_buf, sem_s0)
    g_wait(rows_buf1, sem_g1)
    s_start(abase - 1, rows_buf1, sem_s1)
    s_wait(rows_buf, sem_s0)
    s_wait(rows_buf1, sem_s1)

    @pl.when(wid < arem)
    def _():
        pltpu.sync_copy(g_hbm.at[cid].at[src_idx(abase)], rows_buf)
        pltpu.sync_copy(rows_buf, out_sp.at[dst_idx(abase)], add=True)

    plsc.subcore_barrier()

    flt = jnp.where(cid == 0, 1.0, 0.0).astype(jnp.float32)
    bvecs = [b_buf[pl.ds(j * L, L)] * flt for j in range(cp // L)]

    @pl.loop(0, rpt // CHUNK)
    def _(k):
        sl = pl.ds(r0 + k * CHUNK, CHUNK)
        pltpu.sync_copy(out_sp.at[sl, :], rows_buf)

        @pl.loop(0, CHUNK // L)
        def _(i):
            dvec = dinv_buf[pl.ds(k * CHUNK + i * L, L)]
            for t in range(L):
                s = dvec[t]
                r = i * L + t
                hr = k * CHUNK + r
                for j in range(cp // L):
                    js = pl.ds(j * L, L)
                    rows_buf[r, js] = (s * (rows_buf[r, js]
                                            + hbuf[hr, js] * flt)
                                       + bvecs[j])

        pltpu.sync_copy(rows_buf, acc_hbm.at[cid, sl, :])


def _tc_combine_body(n, c_out, acc_ref, o_ref):
    s = acc_ref[0] + acc_ref[1]
    o_ref[...] = s[:n, :c_out]


def _sc_compiler_params():
    cp = pltpu.CompilerParams()
    fields = pltpu.CompilerParams.__dataclass_fields__
    if "needs_layout_passes" in fields:
        cp = dataclasses.replace(cp, needs_layout_passes=False)
    if "use_tc_tiling_on_sc" in fields:
        cp = dataclasses.replace(cp, use_tc_tiling_on_sc=False)
    return cp


def kernel(x, edge_index, W, b, u):
    n, d_in = x.shape
    c_out = W.shape[1]
    e = edge_index.shape[1]
    assert e % CHUNK == 0, "edge count must be a multiple of 128"

    cp = ((c_out + L - 1) // L) * L
    n_pad = ((n + 1023) // 1024) * 1024
    erows = e // CHUNK
    idx_cap = (erows // NW + 1) * CHUNK
    hidx_cap = (erows // NSUB + 1) * CHUNK

    w_p = jnp.pad(W, ((0, 0), (0, cp - c_out)))
    b_p = jnp.pad(b, (0, cp - c_out))
    u_r = u.reshape(1, d_in)

    h = pl.pallas_call(
        functools.partial(_tc_matmul_body, n, n_pad),
        out_shape=jax.ShapeDtypeStruct((n_pad, cp), jnp.float32),
    )(x, w_p, w_p.T, u_r)

    sc_fn = pl.kernel(
        functools.partial(_sc_body, n_pad, cp, erows),
        out_type=(jax.ShapeDtypeStruct((NSC, n_pad, cp), jnp.float32),
                  jax.ShapeDtypeStruct((NSC, n_pad, cp), jnp.float32)),
        mesh=plsc.VectorSubcoreMesh(core_axis_name="c", subcore_axis_name="s"),
        compiler_params=_sc_compiler_params(),
        scratch_types=[
            pltpu.VMEM_SHARED((n_pad, cp), jnp.float32),
            pltpu.VMEM_SHARED((n_pad,), jnp.float32),
            pltpu.VMEM((hidx_cap,), jnp.int32),
            pltpu.VMEM((idx_cap,), jnp.int32),
            pltpu.VMEM((idx_cap,), jnp.int32),
            pltpu.VMEM((n_pad // NSUB, cp), jnp.float32),
            pltpu.VMEM((CHUNK, cp), jnp.float32),
            pltpu.VMEM((CHUNK, cp), jnp.float32),
            pltpu.VMEM((n_pad // NSUB,), jnp.float32),
            pltpu.VMEM((n_pad // NSUB,), jnp.float32),
            pltpu.VMEM((n_pad // NSUB,), jnp.float32),
            pltpu.VMEM((CHUNK,), jnp.float32),
            pltpu.VMEM((cp,), jnp.float32),
            pltpu.SemaphoreType.DMA,
            pltpu.SemaphoreType.DMA,
            pltpu.SemaphoreType.DMA,
            pltpu.SemaphoreType.DMA,
            pltpu.SemaphoreType.DMA,
            pltpu.SemaphoreType.DMA,
            pltpu.SemaphoreType.DMA,
            pltpu.SemaphoreType.DMA,
            pltpu.SemaphoreType.DMA,
        ],
    )
    acc, _g = sc_fn(h, edge_index, b_p)

    out = pl.pallas_call(
        functools.partial(_tc_combine_body, n, c_out),
        out_shape=jax.ShapeDtypeStruct((n, c_out), jnp.float32),
    )(acc)

    return out


# speedup vs baseline: 53.3031x; 1.0966x over previous
import dataclasses
import functools

import jax
import jax.numpy as jnp
from jax import lax
from jax.experimental import pallas as pl
from jax.experimental.pallas import tpu as pltpu
from jax.experimental.pallas import tpu_sc as plsc

L = 16
NSUB = 16
NSC = 2
NW = NSC * NSUB
CHUNK = 128


def _tc_matmul_body(n, n_pad, x_ref, w_ref, wt_ref, u_ref, h_ref):
    eps = 1e-12
    w = w_ref[...]
    v = jnp.dot(u_ref[...], w, precision=lax.Precision.HIGHEST,
                preferred_element_type=jnp.float32)
    v = v / (jnp.sqrt(jnp.sum(v * v)) + eps)
    wv = jnp.dot(v, wt_ref[...], precision=lax.Precision.HIGHEST,
                 preferred_element_type=jnp.float32)
    un = wv / (jnp.sqrt(jnp.sum(wv * wv)) + eps)
    sigma = jnp.sum(un * wv)
    h = jnp.dot(x_ref[...], w, precision=lax.Precision.HIGHEST,
                preferred_element_type=jnp.float32)
    h_ref[pl.ds(0, n), :] = h * (1.0 / sigma)
    h_ref[pl.ds(n, n_pad - n), :] = jnp.zeros((n_pad - n, h.shape[1]),
                                              jnp.float32)


def _newton_rsqrt(d):
    y = plsc.bitcast(jnp.int32(0x5F3759DF) - (plsc.bitcast(d, jnp.int32) >> 1),
                     jnp.float32)
    for _ in range(3):
        y = y * (1.5 - 0.5 * d * y * y)
    return y


def _sc_body(n_pad, cp, erows,
             h_hbm, edges_hbm, b_hbm, acc_hbm, g_hbm,
             out_sp, deg_sp,
             dsth_buf, srcbuf, dstbuf, hbuf, rows_buf, rows_buf1,
             deg_buf, dinv_buf, zrow_buf, ones_buf, b_buf,
             sem_dsth, sem_src, sem_dst, sem_h, sem_hist,
             sem_g0, sem_g1, sem_s0, sem_s1):
    cid = lax.axis_index("c")
    sid = lax.axis_index("s")
    wid = cid * NSUB + sid
    rpt = n_pad // NSUB
    r0 = sid * rpt

    hbase, hrem = erows // NSUB, erows % NSUB
    abase, arem = erows // NW, erows % NW
    hrow0 = (sid * hbase + jnp.minimum(sid, hrem)) * CHUNK
    arow0 = (wid * abase + jnp.minimum(wid, arem)) * CHUNK
    n_hist = hbase + jnp.where(sid < hrem, 1, 0)

    h_desc = pltpu.async_copy(h_hbm.at[pl.ds(r0, rpt), :], hbuf, sem_h)
    dsth_desc = pltpu.async_copy(
        edges_hbm.at[1, pl.ds(hrow0, hbase * CHUNK)],
        dsth_buf.at[pl.ds(0, hbase * CHUNK)], sem_dsth)

    @pl.when(sid < hrem)
    def _():
        pltpu.sync_copy(edges_hbm.at[1, pl.ds(hrow0 + hbase * CHUNK, CHUNK)],
                        dsth_buf.at[pl.ds(hbase * CHUNK, CHUNK)])

    zv = jnp.zeros((L,), jnp.float32)

    @pl.loop(0, CHUNK)
    def _(r):
        for j in range(cp // L):
            rows_buf[r, pl.ds(j * L, L)] = zv

    @pl.loop(0, rpt // L)
    def _(i):
        zrow_buf[pl.ds(i * L, L)] = zv

    @pl.loop(0, CHUNK // L)
    def _(i):
        ones_buf[pl.ds(i * L, L)] = zv + 1.0

    for k in range(rpt // CHUNK):
        pltpu.sync_copy(rows_buf, out_sp.at[pl.ds(r0 + k * CHUNK, CHUNK), :])
    pltpu.sync_copy(zrow_buf, deg_sp.at[pl.ds(r0, rpt)])
    pltpu.sync_copy(b_hbm, b_buf)
    dsth_desc.wait()
    plsc.subcore_barrier()

    hist_k = 8

    @pl.loop(0, hist_k)
    def _(c):
        pltpu.async_copy(ones_buf, deg_sp.at[dsth_buf.at[pl.ds(c * CHUNK,
                                                               CHUNK)]],
                         sem_hist, add=True)

    @pl.loop(hist_k, n_hist)
    def _(c):
        pltpu.make_async_copy(ones_buf, deg_sp.at[dsth_buf.at[pl.ds(0, CHUNK)]],
                              sem_hist).wait()
        pltpu.async_copy(ones_buf, deg_sp.at[dsth_buf.at[pl.ds(c * CHUNK,
                                                               CHUNK)]],
                         sem_hist, add=True)

    @pl.loop(0, hist_k)
    def _(c):
        pltpu.make_async_copy(ones_buf, deg_sp.at[dsth_buf.at[pl.ds(0, CHUNK)]],
                              sem_hist).wait()

    plsc.subcore_barrier()

    pltpu.sync_copy(deg_sp.at[pl.ds(r0, rpt)], deg_buf)

    @pl.loop(0, rpt // L)
    def _(i):
        d = deg_buf[pl.ds(i * L, L)] + 1.0
        dinv_buf[pl.ds(i * L, L)] = _newton_rsqrt(d)

    h_desc.wait()

    @pl.loop(0, rpt // L)
    def _(i):
        dvec = dinv_buf[pl.ds(i * L, L)]
        for k in range(L):
            s = dvec[k]
            r = i * L + k
            for j in range(cp // L):
                hbuf[r, pl.ds(j * L, L)] = hbuf[r, pl.ds(j * L, L)] * s

    pltpu.sync_copy(hbuf, g_hbm.at[cid].at[pl.ds(r0, rpt), :])

    src_desc = pltpu.async_copy(edges_hbm.at[0, pl.ds(arow0, abase * CHUNK)],
                                srcbuf.at[pl.ds(0, abase * CHUNK)], sem_src)
    dst_desc = pltpu.async_copy(edges_hbm.at[1, pl.ds(arow0, abase * CHUNK)],
                                dstbuf.at[pl.ds(0, abase * CHUNK)], sem_dst)

    @pl.when(wid < arem)
    def _():
        pltpu.sync_copy(edges_hbm.at[0, pl.ds(arow0 + abase * CHUNK, CHUNK)],
                        srcbuf.at[pl.ds(abase * CHUNK, CHUNK)])
        pltpu.sync_copy(edges_hbm.at[1, pl.ds(arow0 + abase * CHUNK, CHUNK)],
                        dstbuf.at[pl.ds(abase * CHUNK, CHUNK)])

    src_desc.wait()
    dst_desc.wait()
    plsc.subcore_barrier()

    def src_idx(c):
        return srcbuf.at[pl.ds(c * CHUNK, CHUNK)]

    def dst_idx(c):
        return dstbuf.at[pl.ds(c * CHUNK, CHUNK)]

    def g_start(c, buf, sem):
        pltpu.async_copy(g_hbm.at[cid].at[src_idx(c)], buf, sem)

    def g_wait(buf, sem):
        pltpu.make_async_copy(g_hbm.at[cid].at[src_idx(0)], buf, sem).wait()

    def s_start(c, buf, sem):
        pltpu.async_copy(buf, out_sp.at[dst_idx(c)], sem, add=True)

    def s_wait(buf, sem):
        pltpu.make_async_copy(buf, out_sp.at[dst_idx(0)], sem).wait()

    g_start(0, rows_buf, sem_g0)
    g_start(1, rows_buf1, sem_g1)

    @pl.loop(0, abase // 2 - 1)
    def _(i):
        c0 = 2 * i
        g_wait(rows_buf, sem_g0)
        s_start(c0, rows_buf, sem_s0)
        g_wait(rows_buf1, sem_g1)
        s_start(c0 + 1, rows_buf1, sem_s1)
        s_wait(rows_buf, sem_s0)
        g_start(c0 + 2, rows_buf, sem_g0)
        s_wait(rows_buf1, sem_s1)
        g_start(c0 + 3, rows_buf1, sem_g1)

    g_wait(rows_buf, sem_g0)
    s_start(abase - 2, rows_buf, sem_s0)
    g_wait(rows_buf1, sem_g1)
    s_start(abase - 1, rows_buf1, sem_s1)
    s_wait(rows_buf, sem_s0)
    s_wait(rows_buf1, sem_s1)

    @pl.when(wid < arem)
    def _():
        pltpu.sync_copy(g_hbm.at[cid].at[src_idx(abase)], rows_buf)
        pltpu.sync_copy(rows_buf, out_sp.at[dst_idx(abase)], add=True)

    plsc.subcore_barrier()

    flt = jnp.where(cid == 0, 1.0, 0.0).astype(jnp.float32)
    bvecs = [b_buf[pl.ds(j * L, L)] * flt for j in range(cp // L)]
    nck = rpt // CHUNK
    bufs = (rows_buf, rows_buf1)
    isems = (sem_g0, sem_g1)
    osems = (sem_s0, sem_s1)

    def p4_in(k, buf, sem):
        return pltpu.async_copy(out_sp.at[pl.ds(r0 + k * CHUNK, CHUNK), :],
                                buf, sem)

    def p4_out(k, buf, sem):
        return pltpu.async_copy(buf, acc_hbm.at[cid,
                                                pl.ds(r0 + k * CHUNK, CHUNK),
                                                :], sem)

    in_d = {0: p4_in(0, bufs[0], isems[0])}
    out_d = {}
    for k in range(nck):
        p = k % 2
        in_d[k].wait()
        if k + 1 < nck:
            if k - 1 >= 0:
                out_d[k - 1].wait()
            in_d[k + 1] = p4_in(k + 1, bufs[(k + 1) % 2], isems[(k + 1) % 2])
        buf = bufs[p]

        @pl.loop(0, CHUNK // L)
        def _(i, k=k, buf=buf):
            dvec = dinv_buf[pl.ds(k * CHUNK + i * L, L)]
            for t in range(L):
                s = dvec[t]
                r = i * L + t
                hr = k * CHUNK + r
                for j in range(cp // L):
                    js = pl.ds(j * L, L)
                    buf[r, js] = (s * (buf[r, js] + hbuf[hr, js] * flt)
                                  + bvecs[j])

        out_d[k] = p4_out(k, buf, osems[p])
    out_d[nck - 2].wait()
    out_d[nck - 1].wait()


def _tc_combine_body(n, c_out, acc_ref, o_ref):
    s = acc_ref[0] + acc_ref[1]
    o_ref[...] = s[:n, :c_out]


def _sc_compiler_params():
    cp = pltpu.CompilerParams()
    fields = pltpu.CompilerParams.__dataclass_fields__
    if "needs_layout_passes" in fields:
        cp = dataclasses.replace(cp, needs_layout_passes=False)
    if "use_tc_tiling_on_sc" in fields:
        cp = dataclasses.replace(cp, use_tc_tiling_on_sc=False)
    return cp


def kernel(x, edge_index, W, b, u):
    n, d_in = x.shape
    c_out = W.shape[1]
    e = edge_index.shape[1]
    assert e % CHUNK == 0, "edge count must be a multiple of 128"

    cp = ((c_out + L - 1) // L) * L
    n_pad = ((n + 1023) // 1024) * 1024
    erows = e // CHUNK
    idx_cap = (erows // NW + 1) * CHUNK
    hidx_cap = (erows // NSUB + 1) * CHUNK

    w_p = jnp.pad(W, ((0, 0), (0, cp - c_out)))
    b_p = jnp.pad(b, (0, cp - c_out))
    u_r = u.reshape(1, d_in)

    h = pl.pallas_call(
        functools.partial(_tc_matmul_body, n, n_pad),
        out_shape=jax.ShapeDtypeStruct((n_pad, cp), jnp.float32),
    )(x, w_p, w_p.T, u_r)

    sc_fn = pl.kernel(
        functools.partial(_sc_body, n_pad, cp, erows),
        out_type=(jax.ShapeDtypeStruct((NSC, n_pad, cp), jnp.float32),
                  jax.ShapeDtypeStruct((NSC, n_pad, cp), jnp.float32)),
        mesh=plsc.VectorSubcoreMesh(core_axis_name="c", subcore_axis_name="s"),
        compiler_params=_sc_compiler_params(),
        scratch_types=[
            pltpu.VMEM_SHARED((n_pad, cp), jnp.float32),
            pltpu.VMEM_SHARED((n_pad,), jnp.float32),
            pltpu.VMEM((hidx_cap,), jnp.int32),
            pltpu.VMEM((idx_cap,), jnp.int32),
            pltpu.VMEM((idx_cap,), jnp.int32),
            pltpu.VMEM((n_pad // NSUB, cp), jnp.float32),
            pltpu.VMEM((CHUNK, cp), jnp.float32),
            pltpu.VMEM((CHUNK, cp), jnp.float32),
            pltpu.VMEM((n_pad // NSUB,), jnp.float32),
            pltpu.VMEM((n_pad // NSUB,), jnp.float32),
            pltpu.VMEM((n_pad // NSUB,), jnp.float32),
            pltpu.VMEM((CHUNK,), jnp.float32),
            pltpu.VMEM((cp,), jnp.float32),
            pltpu.SemaphoreType.DMA,
            pltpu.SemaphoreType.DMA,
            pltpu.SemaphoreType.DMA,
            pltpu.SemaphoreType.DMA,
            pltpu.SemaphoreType.DMA,
            pltpu.SemaphoreType.DMA,
            pltpu.SemaphoreType.DMA,
            pltpu.SemaphoreType.DMA,
            pltpu.SemaphoreType.DMA,
        ],
    )
    acc, _g = sc_fn(h, edge_index, b_p)

    out = pl.pallas_call(
        functools.partial(_tc_combine_body, n, c_out),
        out_shape=jax.ShapeDtypeStruct((n, c_out), jnp.float32),
    )(acc)

    return out


# speedup vs baseline: 62.9672x; 1.1813x over previous
import dataclasses
import functools

import jax
import jax.numpy as jnp
from jax import lax
from jax.experimental import pallas as pl
from jax.experimental.pallas import tpu as pltpu
from jax.experimental.pallas import tpu_sc as plsc

L = 16
NSUB = 16
NSC = 2
NW = NSC * NSUB
CHUNK = 128


def _tc_matmul_body(n, n_pad, x_ref, w_ref, wt_ref, u_ref, h_ref):
    eps = 1e-12
    w = w_ref[...]
    v = jnp.dot(u_ref[...], w, precision=lax.Precision.HIGHEST,
                preferred_element_type=jnp.float32)
    v = v / (jnp.sqrt(jnp.sum(v * v)) + eps)
    wv = jnp.dot(v, wt_ref[...], precision=lax.Precision.HIGHEST,
                 preferred_element_type=jnp.float32)
    un = wv / (jnp.sqrt(jnp.sum(wv * wv)) + eps)
    sigma = jnp.sum(un * wv)
    h = jnp.dot(x_ref[...], w, precision=lax.Precision.HIGHEST,
                preferred_element_type=jnp.float32)
    h_ref[pl.ds(0, n), :] = h * (1.0 / sigma)
    h_ref[pl.ds(n, n_pad - n), :] = jnp.zeros((n_pad - n, h.shape[1]),
                                              jnp.float32)


def _newton_rsqrt(d):
    y = plsc.bitcast(jnp.int32(0x5F3759DF) - (plsc.bitcast(d, jnp.int32) >> 1),
                     jnp.float32)
    for _ in range(3):
        y = y * (1.5 - 0.5 * d * y * y)
    return y


def _sc_body(n_pad, cp, erows,
             h_hbm, edges_hbm, b_hbm, acc_hbm, g_hbm,
             out_sp, deg_sp,
             dsth_buf, srcbuf, dstbuf, hbuf, rows_buf, rows_buf1,
             rows_buf2, rows_buf3,
             deg_buf, dinv_buf, zrow_buf, ones_buf, b_buf,
             sem_dsth, sem_src, sem_dst, sem_h, sem_hist,
             sem_g0, sem_g1, sem_g2, sem_g3,
             sem_s0, sem_s1, sem_s2, sem_s3):
    cid = lax.axis_index("c")
    sid = lax.axis_index("s")
    wid = cid * NSUB + sid
    rpt = n_pad // NSUB
    r0 = sid * rpt

    hbase, hrem = erows // NSUB, erows % NSUB
    abase, arem = erows // NW, erows % NW
    hrow0 = (sid * hbase + jnp.minimum(sid, hrem)) * CHUNK
    arow0 = (wid * abase + jnp.minimum(wid, arem)) * CHUNK
    n_hist = hbase + jnp.where(sid < hrem, 1, 0)

    h_desc = pltpu.async_copy(h_hbm.at[pl.ds(r0, rpt), :], hbuf, sem_h)
    dsth_desc = pltpu.async_copy(
        edges_hbm.at[1, pl.ds(hrow0, hbase * CHUNK)],
        dsth_buf.at[pl.ds(0, hbase * CHUNK)], sem_dsth)

    @pl.when(sid < hrem)
    def _():
        pltpu.sync_copy(edges_hbm.at[1, pl.ds(hrow0 + hbase * CHUNK, CHUNK)],
                        dsth_buf.at[pl.ds(hbase * CHUNK, CHUNK)])

    zv = jnp.zeros((L,), jnp.float32)

    @pl.loop(0, CHUNK)
    def _(r):
        for j in range(cp // L):
            rows_buf[r, pl.ds(j * L, L)] = zv

    @pl.loop(0, rpt // L)
    def _(i):
        zrow_buf[pl.ds(i * L, L)] = zv

    @pl.loop(0, CHUNK // L)
    def _(i):
        ones_buf[pl.ds(i * L, L)] = zv + 1.0

    for k in range(rpt // CHUNK):
        pltpu.sync_copy(rows_buf, out_sp.at[pl.ds(r0 + k * CHUNK, CHUNK), :])
    pltpu.sync_copy(zrow_buf, deg_sp.at[pl.ds(r0, rpt)])
    pltpu.sync_copy(b_hbm, b_buf)
    dsth_desc.wait()
    plsc.subcore_barrier()

    hist_k = 8

    @pl.loop(0, hist_k)
    def _(c):
        pltpu.async_copy(ones_buf, deg_sp.at[dsth_buf.at[pl.ds(c * CHUNK,
                                                               CHUNK)]],
                         sem_hist, add=True)

    @pl.loop(hist_k, n_hist)
    def _(c):
        pltpu.make_async_copy(ones_buf, deg_sp.at[dsth_buf.at[pl.ds(0, CHUNK)]],
                              sem_hist).wait()
        pltpu.async_copy(ones_buf, deg_sp.at[dsth_buf.at[pl.ds(c * CHUNK,
                                                               CHUNK)]],
                         sem_hist, add=True)

    @pl.loop(0, hist_k)
    def _(c):
        pltpu.make_async_copy(ones_buf, deg_sp.at[dsth_buf.at[pl.ds(0, CHUNK)]],
                              sem_hist).wait()

    plsc.subcore_barrier()

    pltpu.sync_copy(deg_sp.at[pl.ds(r0, rpt)], deg_buf)

    @pl.loop(0, rpt // L)
    def _(i):
        d = deg_buf[pl.ds(i * L, L)] + 1.0
        dinv_buf[pl.ds(i * L, L)] = _newton_rsqrt(d)

    h_desc.wait()

    @pl.loop(0, rpt // L)
    def _(i):
        dvec = dinv_buf[pl.ds(i * L, L)]
        for k in range(L):
            s = dvec[k]
            r = i * L + k
            for j in range(cp // L):
                hbuf[r, pl.ds(j * L, L)] = hbuf[r, pl.ds(j * L, L)] * s

    pltpu.sync_copy(hbuf, g_hbm.at[cid].at[pl.ds(r0, rpt), :])

    src_desc = pltpu.async_copy(edges_hbm.at[0, pl.ds(arow0, abase * CHUNK)],
                                srcbuf.at[pl.ds(0, abase * CHUNK)], sem_src)
    dst_desc = pltpu.async_copy(edges_hbm.at[1, pl.ds(arow0, abase * CHUNK)],
                                dstbuf.at[pl.ds(0, abase * CHUNK)], sem_dst)

    @pl.when(wid < arem)
    def _():
        pltpu.sync_copy(edges_hbm.at[0, pl.ds(arow0 + abase * CHUNK, CHUNK)],
                        srcbuf.at[pl.ds(abase * CHUNK, CHUNK)])
        pltpu.sync_copy(edges_hbm.at[1, pl.ds(arow0 + abase * CHUNK, CHUNK)],
                        dstbuf.at[pl.ds(abase * CHUNK, CHUNK)])

    src_desc.wait()
    dst_desc.wait()
    plsc.subcore_barrier()

    def src_idx(c):
        return srcbuf.at[pl.ds(c * CHUNK, CHUNK)]

    def dst_idx(c):
        return dstbuf.at[pl.ds(c * CHUNK, CHUNK)]

    def g_start(c, buf, sem):
        pltpu.async_copy(g_hbm.at[cid].at[src_idx(c)], buf, sem)

    def g_wait(buf, sem):
        pltpu.make_async_copy(g_hbm.at[cid].at[src_idx(0)], buf, sem).wait()

    def s_start(c, buf, sem):
        pltpu.async_copy(buf, out_sp.at[dst_idx(c)], sem, add=True)

    def s_wait(buf, sem):
        pltpu.make_async_copy(buf, out_sp.at[dst_idx(0)], sem).wait()

    abufs = (rows_buf, rows_buf1, rows_buf2, rows_buf3)
    agsems = (sem_g0, sem_g1, sem_g2, sem_g3)
    assems = (sem_s0, sem_s1, sem_s2, sem_s3)
    nb = 4
    for p in range(nb):
        g_start(p, abufs[p], agsems[p])

    @pl.loop(0, abase // nb - 1)
    def _(i):
        c0 = nb * i
        for p in range(nb):
            g_wait(abufs[p], agsems[p])
            s_start(c0 + p, abufs[p], assems[p])
        for p in range(nb):
            s_wait(abufs[p], assems[p])
            g_start(c0 + nb + p, abufs[p], agsems[p])

    it = abase // nb - 1
    gdone = nb * (it + 1)
    sdone = nb * it
    for p in range(nb):
        g_wait(abufs[p], agsems[p])
        s_start(sdone + p, abufs[p], assems[p])
    rem2 = abase - gdone
    for p in range(rem2):
        s_wait(abufs[p], assems[p])
        g_start(gdone + p, abufs[p], agsems[p])
    for p in range(rem2):
        g_wait(abufs[p], agsems[p])
        s_start(gdone + p, abufs[p], assems[p])
    for p in range(nb):
        s_wait(abufs[p], assems[p])

    @pl.when(wid < arem)
    def _():
        pltpu.sync_copy(g_hbm.at[cid].at[src_idx(abase)], rows_buf)
        pltpu.sync_copy(rows_buf, out_sp.at[dst_idx(abase)], add=True)

    plsc.subcore_barrier()

    flt = jnp.where(cid == 0, 1.0, 0.0).astype(jnp.float32)
    bvecs = [b_buf[pl.ds(j * L, L)] * flt for j in range(cp // L)]
    nck = rpt // CHUNK
    bufs = (rows_buf, rows_buf1)
    isems = (sem_g0, sem_g1)
    osems = (sem_s0, sem_s1)

    def p4_in(k, buf, sem):
        return pltpu.async_copy(out_sp.at[pl.ds(r0 + k * CHUNK, CHUNK), :],
                                buf, sem)

    def p4_out(k, buf, sem):
        return pltpu.async_copy(buf, acc_hbm.at[cid,
                                                pl.ds(r0 + k * CHUNK, CHUNK),
                                                :], sem)

    in_d = {0: p4_in(0, bufs[0], isems[0])}
    out_d = {}
    for k in range(nck):
        p = k % 2
        in_d[k].wait()
        if k + 1 < nck:
            if k - 1 >= 0:
                out_d[k - 1].wait()
            in_d[k + 1] = p4_in(k + 1, bufs[(k + 1) % 2], isems[(k + 1) % 2])
        buf = bufs[p]

        @pl.loop(0, CHUNK // L)
        def _(i, k=k, buf=buf):
            dvec = dinv_buf[pl.ds(k * CHUNK + i * L, L)]
            for t in range(L):
                s = dvec[t]
                r = i * L + t
                hr = k * CHUNK + r
                for j in range(cp // L):
                    js = pl.ds(j * L, L)
                    buf[r, js] = (s * (buf[r, js] + hbuf[hr, js] * flt)
                                  + bvecs[j])

        out_d[k] = p4_out(k, buf, osems[p])
    out_d[nck - 2].wait()
    out_d[nck - 1].wait()


def _tc_combine_body(n, c_out, acc_ref, o_ref):
    s = acc_ref[0] + acc_ref[1]
    o_ref[...] = s[:n, :c_out]


def _sc_compiler_params():
    cp = pltpu.CompilerParams()
    fields = pltpu.CompilerParams.__dataclass_fields__
    if "needs_layout_passes" in fields:
        cp = dataclasses.replace(cp, needs_layout_passes=False)
    if "use_tc_tiling_on_sc" in fields:
        cp = dataclasses.replace(cp, use_tc_tiling_on_sc=False)
    return cp


def kernel(x, edge_index, W, b, u):
    n, d_in = x.shape
    c_out = W.shape[1]
    e = edge_index.shape[1]
    assert e % CHUNK == 0, "edge count must be a multiple of 128"

    cp = ((c_out + L - 1) // L) * L
    n_pad = ((n + 1023) // 1024) * 1024
    erows = e // CHUNK
    idx_cap = (erows // NW + 1) * CHUNK
    hidx_cap = (erows // NSUB + 1) * CHUNK

    w_p = jnp.pad(W, ((0, 0), (0, cp - c_out)))
    b_p = jnp.pad(b, (0, cp - c_out))
    u_r = u.reshape(1, d_in)

    h = pl.pallas_call(
        functools.partial(_tc_matmul_body, n, n_pad),
        out_shape=jax.ShapeDtypeStruct((n_pad, cp), jnp.float32),
    )(x, w_p, w_p.T, u_r)

    sc_fn = pl.kernel(
        functools.partial(_sc_body, n_pad, cp, erows),
        out_type=(jax.ShapeDtypeStruct((NSC, n_pad, cp), jnp.float32),
                  jax.ShapeDtypeStruct((NSC, n_pad, cp), jnp.float32)),
        mesh=plsc.VectorSubcoreMesh(core_axis_name="c", subcore_axis_name="s"),
        compiler_params=_sc_compiler_params(),
        scratch_types=[
            pltpu.VMEM_SHARED((n_pad, cp), jnp.float32),
            pltpu.VMEM_SHARED((n_pad,), jnp.float32),
            pltpu.VMEM((hidx_cap,), jnp.int32),
            pltpu.VMEM((idx_cap,), jnp.int32),
            pltpu.VMEM((idx_cap,), jnp.int32),
            pltpu.VMEM((n_pad // NSUB, cp), jnp.float32),
            pltpu.VMEM((CHUNK, cp), jnp.float32),
            pltpu.VMEM((CHUNK, cp), jnp.float32),
            pltpu.VMEM((CHUNK, cp), jnp.float32),
            pltpu.VMEM((CHUNK, cp), jnp.float32),
            pltpu.VMEM((n_pad // NSUB,), jnp.float32),
            pltpu.VMEM((n_pad // NSUB,), jnp.float32),
            pltpu.VMEM((n_pad // NSUB,), jnp.float32),
            pltpu.VMEM((CHUNK,), jnp.float32),
            pltpu.VMEM((cp,), jnp.float32),
            pltpu.SemaphoreType.DMA,
            pltpu.SemaphoreType.DMA,
            pltpu.SemaphoreType.DMA,
            pltpu.SemaphoreType.DMA,
            pltpu.SemaphoreType.DMA,
            pltpu.SemaphoreType.DMA,
            pltpu.SemaphoreType.DMA,
            pltpu.SemaphoreType.DMA,
            pltpu.SemaphoreType.DMA,
            pltpu.SemaphoreType.DMA,
            pltpu.SemaphoreType.DMA,
            pltpu.SemaphoreType.DMA,
            pltpu.SemaphoreType.DMA,
        ],
    )
    acc, _g = sc_fn(h, edge_index, b_p)

    out = pl.pallas_call(
        functools.partial(_tc_combine_body, n, c_out),
        out_shape=jax.ShapeDtypeStruct((n, c_out), jnp.float32),
    )(acc)

    return out
